# baseline (speedup 1.0000x reference)
import functools

import jax
import jax.numpy as jnp
from jax import lax
from jax.experimental import pallas as pl
from jax.experimental.pallas import tpu as pltpu
from jax.experimental.pallas import tpu_sc as plsc

_L = 16


def _row_argmax(buf, n):
    iota = lax.iota(jnp.int32, _L)
    neg = jnp.full((_L,), -jnp.inf, jnp.float32)
    zero = jnp.zeros((_L,), jnp.int32)
    n_chunks = n // _L
    n_acc = 8
    unroll = 2
    n_iter = n_chunks // (n_acc * unroll)

    def body(i, carry):
        carry = list(carry)
        for u in range(unroll):
            g = i * unroll + u
            base = g * (n_acc * _L)
            gg = jnp.broadcast_to(g, (_L,))
            for j in range(n_acc):
                mv, mi = carry[2 * j], carry[2 * j + 1]
                v = buf[pl.ds(base + j * _L, _L)]
                gt = v > mv
                carry[2 * j] = jnp.where(gt, v, mv)
                carry[2 * j + 1] = jnp.where(gt, gg, mi)
        return tuple(carry)

    carry = lax.fori_loop(0, n_iter, body, (neg, zero) * n_acc)

    mv = carry[0]
    mi = carry[1] * (n_acc * _L) + iota
    for j in range(1, n_acc):
        vb = carry[2 * j]
        ib = carry[2 * j + 1] * (n_acc * _L) + j * _L + iota
        take_a = (mv > vb) | ((mv == vb) & (mi < ib))
        mv = jnp.where(take_a, mv, vb)
        mi = jnp.where(take_a, mi, ib)

    bv, bi = mv[0], mi[0]
    for l in range(1, _L):
        v, ix = mv[l], mi[l]
        take = (v > bv) | ((v == bv) & (ix < bi))
        bv = jnp.where(take, v, bv)
        bi = jnp.where(take, ix, bi)
    return bi


def _sc_argmax(probas, n_rows):
    B, T, V = probas.shape
    info = plsc.get_sparse_core_info()
    NC, NS = info.num_cores, info.num_subcores
    NW = NC * NS
    rows_per_w = n_rows // NW
    grp = 8 // rows_per_w
    mesh = plsc.VectorSubcoreMesh(core_axis_name="c", subcore_axis_name="s")

    @functools.partial(
        pl.kernel,
        mesh=mesh,
        out_type=jax.ShapeDtypeStruct((n_rows,), jnp.int32),
        scratch_types=[pltpu.VMEM((V,), jnp.float32) for _ in range(rows_per_w)]
        + [pltpu.VMEM((_L,), jnp.int32),
           pltpu.VMEM(((grp - 1) * _L,), jnp.int32),
           pltpu.VMEM_SHARED((NS * _L,), jnp.int32)]
        + [pltpu.SemaphoreType.DMA for _ in range(rows_per_w)],
    )
    def k(probas_hbm, cand_hbm, *scratch):
        bufs = scratch[:rows_per_w]
        candbuf, pbuf, shared = scratch[rows_per_w:rows_per_w + 3]
        isems = scratch[rows_per_w + 3:]
        cid = lax.axis_index("c")
        sid = lax.axis_index("s")
        wid = cid * NS + sid
        row0 = wid * rows_per_w

        ins = [
            pltpu.async_copy(probas_hbm.at[row0 + r, T - 1], bufs[r], isems[r])
            for r in range(rows_per_w)
        ]
        bests = []
        for r in range(rows_per_w):
            ins[r].wait()
            bests.append(_row_argmax(bufs[r], V))

        iota = lax.iota(jnp.int32, _L)
        q = sid % grp
        lane0 = q * rows_per_w
        v = jnp.zeros((_L,), jnp.int32)
        for r in range(rows_per_w):
            v = jnp.where(iota == lane0 + r, bests[r], v)
        candbuf[pl.ds(0, _L)] = v
        pltpu.sync_copy(candbuf, shared.at[pl.ds(sid * _L, _L)])
        plsc.subcore_barrier()

        @pl.when(q == 0)
        def _():
            pltpu.sync_copy(
                shared.at[pl.ds((sid + 1) * _L, (grp - 1) * _L)], pbuf)
            merged = v
            for p in range(1, grp):
                vp = pbuf[pl.ds((p - 1) * _L, _L)]
                mask = (iota >= p * rows_per_w) & (iota < (p + 1) * rows_per_w)
                merged = jnp.where(mask, vp, merged)
            candbuf[pl.ds(0, _L)] = merged
            grp_base = (wid // grp) * 8
            pltpu.sync_copy(candbuf.at[pl.ds(0, 8)],
                            cand_hbm.at[pl.ds(grp_base, 8)])

    return k(probas)


def _dist_copy_and_upper_argmax(probas, row_split):
    B, T, V = probas.shape
    blk = 8
    nblk = B // blk
    n_hi = B - row_split

    def body(in_ref, dist_ref, cand_ref, vbuf, vcand, insem, outsem, csem):
        ins = []
        for c in range(nblk):
            cp = pltpu.make_async_copy(
                in_ref.at[pl.ds(c * blk, blk), T - 1],
                vbuf.at[pl.ds(c * blk, blk)], insem)
            cp.start()
            ins.append(cp)
        outs = []
        iota2d = lax.broadcasted_iota(jnp.int32, (blk, V), 1)
        big = jnp.int32(2**30)
        for c in range(nblk):
            ins[c].wait()
            cp = pltpu.make_async_copy(
                vbuf.at[pl.ds(c * blk, blk)],
                dist_ref.at[pl.ds(c * blk, blk)], outsem)
            cp.start()
            outs.append(cp)
            if c * blk >= row_split:
                x = vbuf[pl.ds(c * blk, blk), :]
                mx = jnp.max(x, axis=1, keepdims=True)
                mi = jnp.min(jnp.where(x == mx, iota2d, big), axis=1,
                             keepdims=True)
                vcand[pl.ds(c * blk - row_split, blk), :] = mi
        ccp = pltpu.make_async_copy(vcand, cand_ref.at[pl.ds(row_split, n_hi)], csem)
        ccp.start()
        ccp.wait()
        for cp in outs:
            cp.wait()

    return pl.pallas_call(
        body,
        in_specs=[pl.BlockSpec(memory_space=pltpu.MemorySpace.HBM)],
        out_specs=[pl.BlockSpec(memory_space=pltpu.MemorySpace.HBM),
                   pl.BlockSpec(memory_space=pltpu.MemorySpace.HBM)],
        scratch_shapes=[pltpu.VMEM((B, V), jnp.float32),
                        pltpu.VMEM((n_hi, 1), jnp.int32),
                        pltpu.SemaphoreType.DMA,
                        pltpu.SemaphoreType.DMA,
                        pltpu.SemaphoreType.DMA],
        out_shape=[jax.ShapeDtypeStruct((B, V), jnp.float32),
                   jax.ShapeDtypeStruct((B, 1), jnp.int32)],
    )(probas)


@jax.jit
def _candidate(probas):
    B = probas.shape[0]
    split = B // 4
    dist, cand_hi = _dist_copy_and_upper_argmax(probas, split)
    cand_lo = _sc_argmax(probas, split)
    candidate = lax.dynamic_update_slice(
        cand_hi, cand_lo.reshape(split, 1), (0, 0))
    return candidate, dist


def kernel(probas, greedy):
    del greedy
    return _candidate(probas)


# speedup vs baseline: 1.0889x; 1.0889x over previous
import functools

import jax
import jax.numpy as jnp
from jax import lax
from jax.experimental import pallas as pl
from jax.experimental.pallas import tpu as pltpu
from jax.experimental.pallas import tpu_sc as plsc

_L = 16


def _row_argmax(buf, n):
    iota = lax.iota(jnp.int32, _L)
    neg = jnp.full((_L,), -jnp.inf, jnp.float32)
    zero = jnp.zeros((_L,), jnp.int32)
    n_chunks = n // _L
    n_acc = 8
    unroll = 2
    n_iter = n_chunks // (n_acc * unroll)

    def body(i, carry):
        carry = list(carry)
        for u in range(unroll):
            g = i * unroll + u
            base = g * (n_acc * _L)
            gg = jnp.broadcast_to(g, (_L,))
            for j in range(n_acc):
                mv, mi = carry[2 * j], carry[2 * j + 1]
                v = buf[pl.ds(base + j * _L, _L)]
                gt = v > mv
                carry[2 * j] = jnp.where(gt, v, mv)
                carry[2 * j + 1] = jnp.where(gt, gg, mi)
        return tuple(carry)

    carry = lax.fori_loop(0, n_iter, body, (neg, zero) * n_acc)

    mv = carry[0]
    mi = carry[1] * (n_acc * _L) + iota
    for j in range(1, n_acc):
        vb = carry[2 * j]
        ib = carry[2 * j + 1] * (n_acc * _L) + j * _L + iota
        take_a = (mv > vb) | ((mv == vb) & (mi < ib))
        mv = jnp.where(take_a, mv, vb)
        mi = jnp.where(take_a, mi, ib)

    bv, bi = mv[0], mi[0]
    for l in range(1, _L):
        v, ix = mv[l], mi[l]
        take = (v > bv) | ((v == bv) & (ix < bi))
        bv = jnp.where(take, v, bv)
        bi = jnp.where(take, ix, bi)
    return bi


def _sc_argmax(probas, n_rows):
    B, T, V = probas.shape
    info = plsc.get_sparse_core_info()
    NC, NS = info.num_cores, info.num_subcores
    NW = NC * NS
    rows_per_w = n_rows // NW
    grp = 8 // rows_per_w
    mesh = plsc.VectorSubcoreMesh(core_axis_name="c", subcore_axis_name="s")

    @functools.partial(
        pl.kernel,
        mesh=mesh,
        out_type=jax.ShapeDtypeStruct((n_rows,), jnp.int32),
        scratch_types=[pltpu.VMEM((V,), jnp.float32) for _ in range(rows_per_w)]
        + [pltpu.VMEM((_L,), jnp.int32),
           pltpu.VMEM(((grp - 1) * _L,), jnp.int32),
           pltpu.VMEM_SHARED((NS * _L,), jnp.int32)]
        + [pltpu.SemaphoreType.DMA for _ in range(rows_per_w)],
    )
    def k(probas_hbm, cand_hbm, *scratch):
        bufs = scratch[:rows_per_w]
        candbuf, pbuf, shared = scratch[rows_per_w:rows_per_w + 3]
        isems = scratch[rows_per_w + 3:]
        cid = lax.axis_index("c")
        sid = lax.axis_index("s")
        wid = cid * NS + sid
        row0 = wid * rows_per_w

        ins = [
            pltpu.async_copy(probas_hbm.at[row0 + r, T - 1], bufs[r], isems[r])
            for r in range(rows_per_w)
        ]
        bests = []
        for r in range(rows_per_w):
            ins[r].wait()
            bests.append(_row_argmax(bufs[r], V))

        iota = lax.iota(jnp.int32, _L)
        q = sid % grp
        lane0 = q * rows_per_w
        v = jnp.zeros((_L,), jnp.int32)
        for r in range(rows_per_w):
            v = jnp.where(iota == lane0 + r, bests[r], v)
        candbuf[pl.ds(0, _L)] = v
        pltpu.sync_copy(candbuf, shared.at[pl.ds(sid * _L, _L)])
        plsc.subcore_barrier()

        @pl.when(q == 0)
        def _():
            pltpu.sync_copy(
                shared.at[pl.ds((sid + 1) * _L, (grp - 1) * _L)], pbuf)
            merged = v
            for p in range(1, grp):
                vp = pbuf[pl.ds((p - 1) * _L, _L)]
                mask = (iota >= p * rows_per_w) & (iota < (p + 1) * rows_per_w)
                merged = jnp.where(mask, vp, merged)
            candbuf[pl.ds(0, _L)] = merged
            grp_base = (wid // grp) * 8
            pltpu.sync_copy(candbuf.at[pl.ds(0, 8)],
                            cand_hbm.at[pl.ds(grp_base, 8)])

    return k(probas)


def _dist_copy_and_upper_argmax(probas, row_split):
    B, T, V = probas.shape
    blk = 8
    nblk = B // blk
    n_hi = B - row_split

    def body(in_ref, dist_ref, cand_ref, vbuf, vcand, insem, outsem, csem):
        ins = []
        for c in range(nblk):
            cp = pltpu.make_async_copy(
                in_ref.at[pl.ds(c * blk, blk), T - 1],
                vbuf.at[pl.ds(c * blk, blk)], insem)
            cp.start()
            ins.append(cp)
        outs = []
        iota2d = lax.broadcasted_iota(jnp.int32, (blk, V), 1)
        big = jnp.int32(2**30)
        for c in range(nblk):
            ins[c].wait()
            cp = pltpu.make_async_copy(
                vbuf.at[pl.ds(c * blk, blk)],
                dist_ref.at[pl.ds(c * blk, blk)], outsem)
            cp.start()
            outs.append(cp)
            if c * blk >= row_split:
                x = vbuf[pl.ds(c * blk, blk), :]
                mx = jnp.max(x, axis=1, keepdims=True)
                mi = jnp.min(jnp.where(x == mx, iota2d, big), axis=1,
                             keepdims=True)
                vcand[pl.ds(c * blk - row_split, blk), :] = mi
        ccp = pltpu.make_async_copy(vcand, cand_ref, csem)
        ccp.start()
        ccp.wait()
        for cp in outs:
            cp.wait()

    return pl.pallas_call(
        body,
        in_specs=[pl.BlockSpec(memory_space=pltpu.MemorySpace.HBM)],
        out_specs=[pl.BlockSpec(memory_space=pltpu.MemorySpace.HBM),
                   pl.BlockSpec(memory_space=pltpu.MemorySpace.HBM)],
        scratch_shapes=[pltpu.VMEM((B, V), jnp.float32),
                        pltpu.VMEM((n_hi, 1), jnp.int32),
                        pltpu.SemaphoreType.DMA,
                        pltpu.SemaphoreType.DMA,
                        pltpu.SemaphoreType.DMA],
        out_shape=[jax.ShapeDtypeStruct((B, V), jnp.float32),
                   jax.ShapeDtypeStruct((n_hi, 1), jnp.int32)],
    )(probas)


@jax.jit
def _candidate(probas):
    B = probas.shape[0]
    split = B // 4
    dist, cand_hi = _dist_copy_and_upper_argmax(probas, split)
    cand_lo = _sc_argmax(probas, split)
    candidate = jnp.concatenate(
        [cand_lo, cand_hi.reshape(B - split)]).reshape(B, 1)
    return candidate, dist


def kernel(probas, greedy):
    del greedy
    return _candidate(probas)
